# Initial kernel scaffold; baseline (speedup 1.0000x reference)
import jax
import jax.numpy as jnp
from jax.experimental import pallas as pl


def kernel(features, labels, preds):
    raise NotImplementedError("write your pallas kernel here")


# speedup vs baseline: 63.0204x; 63.0204x over previous
import jax
import jax.numpy as jnp
from jax import lax
from jax.experimental import pallas as pl

N = 8192
D = 2048
K = 1024
RB = 512
NBLK = N // RB
PB = 256
F32 = jnp.float32


def _centers_body(lab_ref, f_ref, sum_ref, cntc_ref, cntr_ref):
    i = pl.program_id(0)

    @pl.when(i == 0)
    def _():
        sum_ref[...] = jnp.zeros_like(sum_ref)
        cntc_ref[...] = jnp.zeros_like(cntc_ref)
        cntr_ref[...] = jnp.zeros_like(cntr_ref)

    lab = lab_ref[...].astype(jnp.int32)
    ohT = (lax.broadcasted_iota(jnp.int32, (K, RB), 0) == lab).astype(F32)
    sum_ref[...] += lax.dot_general(
        ohT, f_ref[...], (((1,), (0,)), ((), ())), preferred_element_type=F32)
    cntc_ref[...] += jnp.sum(ohT, axis=1, keepdims=True)
    ones = jnp.ones((1, RB), F32)
    cntr_ref[...] += lax.dot_general(
        ones, ohT, (((1,), (1,)), ((), ())), preferred_element_type=F32)


def _norm_body(sum_ref, cntc_ref, p_ref, xxc_ref, xxr_ref):
    cnt = cntc_ref[...]
    c = sum_ref[...] / jnp.where(cnt > 0, cnt, 1.0)
    nrm = jnp.maximum(jnp.sqrt(jnp.sum(c * c, axis=1, keepdims=True)), 1e-12)
    p = c / nrm
    p_ref[...] = p
    psq = p * p
    xxc_ref[...] = jnp.sum(psq, axis=1, keepdims=True)
    ones = jnp.ones((1, D), F32)
    xxr_ref[...] = lax.dot_general(
        ones, psq, (((1,), (1,)), ((), ())), preferred_element_type=F32)


def _pair_body(pi_ref, pj_ref, xxi_ref, xxj_ref, ci_ref, cj_ref, cfull_ref,
               acc_ref):
    i = pl.program_id(0)
    j = pl.program_id(1)
    lane = lax.broadcasted_iota(jnp.int32, (1, 128), 1)

    @pl.when((i == 0) & (j == 0))
    def _():
        npres = jnp.sum((cfull_ref[...] > 0).astype(F32))
        init = jnp.where(lane == 2, jnp.inf,
                         jnp.where(lane == 3, -jnp.inf,
                                   jnp.where(lane == 4, npres, 0.0)))
        acc_ref[...] = init

    g = lax.dot_general(pi_ref[...], pj_ref[...],
                        (((1,), (1,)), ((), ())), preferred_element_type=F32)
    d = xxi_ref[...] + xxj_ref[...] - 2.0 * g
    dist = jnp.sqrt(jnp.maximum(d, 1e-12))
    gi = lax.broadcasted_iota(jnp.int32, (PB, PB), 0) + i * PB
    gj = lax.broadcasted_iota(jnp.int32, (PB, PB), 1) + j * PB
    wm = jnp.abs(gi - gj).astype(F32)
    mask = (gj > gi) & (ci_ref[...] > 0) & (cj_ref[...] > 0)
    s0 = jnp.sum(jnp.where(mask, dist, 0.0))
    s1 = jnp.sum(jnp.where(mask, dist * wm, 0.0))
    wmn = jnp.min(jnp.where(mask, wm, jnp.inf))
    wmx = jnp.max(jnp.where(mask, wm, -jnp.inf))
    r = acc_ref[...]
    r = r + jnp.where(lane == 0, s0, 0.0) + jnp.where(lane == 1, s1, 0.0)
    r = jnp.where(lane == 2, jnp.minimum(r, wmn), r)
    r = jnp.where(lane == 3, jnp.maximum(r, wmx), r)
    acc_ref[...] = r


def _tight_body(lab_ref, pred_ref, f_ref, p_ref, acc_ref):
    i = pl.program_id(0)
    lane = lax.broadcasted_iota(jnp.int32, (1, 128), 1)

    @pl.when(i == 0)
    def _():
        acc_ref[...] = jnp.zeros_like(acc_ref)

    lab_f = lab_ref[...]
    lab = lab_f.astype(jnp.int32)
    ohT = (lax.broadcasted_iota(jnp.int32, (K, RB), 0) == lab).astype(F32)
    fc = lax.dot_general(ohT, p_ref[...],
                         (((0,), (0,)), ((), ())), preferred_element_type=F32)
    dif = f_ref[...] - fc
    t = jnp.sum(dif * dif, axis=1, keepdims=True)
    mask = t > 0
    s = jnp.sqrt(jnp.where(mask, t, 1.0))
    ssum = jnp.sum(jnp.where(mask, s, 0.0))
    scnt = jnp.sum(mask.astype(F32))
    e = lab_f - pred_ref[...]
    sse = jnp.sum(e * e)
    r = acc_ref[...]
    r = (r + jnp.where(lane == 0, ssum, 0.0)
         + jnp.where(lane == 1, scnt, 0.0)
         + jnp.where(lane == 2, sse, 0.0))
    acc_ref[...] = r


def kernel(features, labels, preds):
    lab3 = labels.reshape(NBLK, 1, RB)
    pred3 = preds.reshape(NBLK, 1, RB)

    center_sum, cntc, cntr = pl.pallas_call(
        _centers_body,
        grid=(NBLK,),
        in_specs=[
            pl.BlockSpec((None, 1, RB), lambda i: (i, 0, 0)),
            pl.BlockSpec((RB, D), lambda i: (i, 0)),
        ],
        out_specs=[
            pl.BlockSpec((K, D), lambda i: (0, 0)),
            pl.BlockSpec((K, 1), lambda i: (0, 0)),
            pl.BlockSpec((1, K), lambda i: (0, 0)),
        ],
        out_shape=[
            jax.ShapeDtypeStruct((K, D), F32),
            jax.ShapeDtypeStruct((K, 1), F32),
            jax.ShapeDtypeStruct((1, K), F32),
        ],
    )(lab3, features)

    p, xxc, xxr = pl.pallas_call(
        _norm_body,
        out_shape=[
            jax.ShapeDtypeStruct((K, D), F32),
            jax.ShapeDtypeStruct((K, 1), F32),
            jax.ShapeDtypeStruct((1, K), F32),
        ],
    )(center_sum, cntc)

    acc = pl.pallas_call(
        _pair_body,
        grid=(K // PB, K // PB),
        in_specs=[
            pl.BlockSpec((PB, D), lambda i, j: (i, 0)),
            pl.BlockSpec((PB, D), lambda i, j: (j, 0)),
            pl.BlockSpec((PB, 1), lambda i, j: (i, 0)),
            pl.BlockSpec((1, PB), lambda i, j: (0, j)),
            pl.BlockSpec((PB, 1), lambda i, j: (i, 0)),
            pl.BlockSpec((1, PB), lambda i, j: (0, j)),
            pl.BlockSpec((1, K), lambda i, j: (0, 0)),
        ],
        out_specs=pl.BlockSpec((1, 128), lambda i, j: (0, 0)),
        out_shape=jax.ShapeDtypeStruct((1, 128), F32),
    )(p, p, xxc, xxr, cntc, cntr, cntr)

    acc2 = pl.pallas_call(
        _tight_body,
        grid=(NBLK,),
        in_specs=[
            pl.BlockSpec((None, 1, RB), lambda i: (i, 0, 0)),
            pl.BlockSpec((None, 1, RB), lambda i: (i, 0, 0)),
            pl.BlockSpec((RB, D), lambda i: (i, 0)),
            pl.BlockSpec((K, D), lambda i: (0, 0)),
        ],
        out_specs=pl.BlockSpec((1, 128), lambda i: (0, 0)),
        out_shape=jax.ShapeDtypeStruct((1, 128), F32),
    )(lab3, pred3, features, p)

    s0 = acc[0, 0]
    s1 = acc[0, 1]
    wmn = acc[0, 2]
    wmx = acc[0, 3]
    npres = acc[0, 4]
    ssum = acc2[0, 0]
    scnt = acc2[0, 1]
    sse = acc2[0, 2]

    n_pairs = npres * (npres - 1.0) * 0.5
    entropy = (s1 - wmn * s0) / wmx / n_pairs
    tight = ssum / jnp.maximum(scnt, 1.0)
    mse = sse / N
    return mse + 0.001 * (tight - entropy)


# speedup vs baseline: 63.1062x; 1.0014x over previous
import jax
import jax.numpy as jnp
from jax import lax
from jax.experimental import pallas as pl

N = 8192
D = 2048
K = 1024
RB = 512
NBLK = N // RB
PB = 256
F32 = jnp.float32


def _centers_body(lab_ref, f_ref, sum_ref, cntc_ref, cntr_ref):
    i = pl.program_id(0)

    @pl.when(i == 0)
    def _():
        sum_ref[...] = jnp.zeros_like(sum_ref)
        cntc_ref[...] = jnp.zeros_like(cntc_ref)
        cntr_ref[...] = jnp.zeros_like(cntr_ref)

    lab = lab_ref[...].astype(jnp.int32)
    ohT = (lax.broadcasted_iota(jnp.int32, (K, RB), 0) == lab).astype(jnp.bfloat16)
    sum_ref[...] += lax.dot_general(
        ohT, f_ref[...].astype(jnp.bfloat16), (((1,), (0,)), ((), ())),
        preferred_element_type=F32)
    ohT_f = ohT.astype(F32)
    cntc_ref[...] += jnp.sum(ohT_f, axis=1, keepdims=True)
    ones = jnp.ones((1, RB), F32)
    cntr_ref[...] += lax.dot_general(
        ones, ohT_f, (((1,), (1,)), ((), ())), preferred_element_type=F32)


def _norm_body(sum_ref, cntc_ref, p_ref, xxc_ref, xxr_ref):
    cnt = cntc_ref[...]
    c = sum_ref[...] / jnp.where(cnt > 0, cnt, 1.0)
    nrm = jnp.maximum(jnp.sqrt(jnp.sum(c * c, axis=1, keepdims=True)), 1e-12)
    p = c / nrm
    p_ref[...] = p
    psq = p * p
    xxc_ref[...] = jnp.sum(psq, axis=1, keepdims=True)
    ones = jnp.ones((1, D), F32)
    xxr_ref[...] = lax.dot_general(
        ones, psq, (((1,), (1,)), ((), ())), preferred_element_type=F32)


def _pair_body(pi_ref, pj_ref, xxi_ref, xxj_ref, ci_ref, cj_ref, cfull_ref,
               acc_ref):
    i = pl.program_id(0)
    j = pl.program_id(1)
    lane = lax.broadcasted_iota(jnp.int32, (1, 128), 1)

    @pl.when((i == 0) & (j == 0))
    def _():
        npres = jnp.sum((cfull_ref[...] > 0).astype(F32))
        init = jnp.where(lane == 2, jnp.inf,
                         jnp.where(lane == 3, -jnp.inf,
                                   jnp.where(lane == 4, npres, 0.0)))
        acc_ref[...] = init

    g = lax.dot_general(pi_ref[...], pj_ref[...],
                        (((1,), (1,)), ((), ())), preferred_element_type=F32)
    d = xxi_ref[...] + xxj_ref[...] - 2.0 * g
    dist = jnp.sqrt(jnp.maximum(d, 1e-12))
    gi = lax.broadcasted_iota(jnp.int32, (PB, PB), 0) + i * PB
    gj = lax.broadcasted_iota(jnp.int32, (PB, PB), 1) + j * PB
    wm = jnp.abs(gi - gj).astype(F32)
    mask = (gj > gi) & (ci_ref[...] > 0) & (cj_ref[...] > 0)
    s0 = jnp.sum(jnp.where(mask, dist, 0.0))
    s1 = jnp.sum(jnp.where(mask, dist * wm, 0.0))
    wmn = jnp.min(jnp.where(mask, wm, jnp.inf))
    wmx = jnp.max(jnp.where(mask, wm, -jnp.inf))
    r = acc_ref[...]
    r = r + jnp.where(lane == 0, s0, 0.0) + jnp.where(lane == 1, s1, 0.0)
    r = jnp.where(lane == 2, jnp.minimum(r, wmn), r)
    r = jnp.where(lane == 3, jnp.maximum(r, wmx), r)
    acc_ref[...] = r


def _tight_body(lab_ref, pred_ref, f_ref, p_ref, acc_ref):
    i = pl.program_id(0)
    lane = lax.broadcasted_iota(jnp.int32, (1, 128), 1)

    @pl.when(i == 0)
    def _():
        acc_ref[...] = jnp.zeros_like(acc_ref)

    lab_f = lab_ref[...]
    lab = lab_f.astype(jnp.int32)
    ohT = (lax.broadcasted_iota(jnp.int32, (K, RB), 0) == lab).astype(jnp.bfloat16)
    fc = lax.dot_general(ohT, p_ref[...].astype(jnp.bfloat16),
                         (((0,), (0,)), ((), ())), preferred_element_type=F32)
    dif = f_ref[...] - fc
    t = jnp.sum(dif * dif, axis=1, keepdims=True)
    mask = t > 0
    s = jnp.sqrt(jnp.where(mask, t, 1.0))
    ssum = jnp.sum(jnp.where(mask, s, 0.0))
    scnt = jnp.sum(mask.astype(F32))
    e = lab_f - pred_ref[...]
    sse = jnp.sum(e * e)
    r = acc_ref[...]
    r = (r + jnp.where(lane == 0, ssum, 0.0)
         + jnp.where(lane == 1, scnt, 0.0)
         + jnp.where(lane == 2, sse, 0.0))
    acc_ref[...] = r


def kernel(features, labels, preds):
    lab3 = labels.reshape(NBLK, 1, RB)
    pred3 = preds.reshape(NBLK, 1, RB)

    center_sum, cntc, cntr = pl.pallas_call(
        _centers_body,
        grid=(NBLK,),
        in_specs=[
            pl.BlockSpec((None, 1, RB), lambda i: (i, 0, 0)),
            pl.BlockSpec((RB, D), lambda i: (i, 0)),
        ],
        out_specs=[
            pl.BlockSpec((K, D), lambda i: (0, 0)),
            pl.BlockSpec((K, 1), lambda i: (0, 0)),
            pl.BlockSpec((1, K), lambda i: (0, 0)),
        ],
        out_shape=[
            jax.ShapeDtypeStruct((K, D), F32),
            jax.ShapeDtypeStruct((K, 1), F32),
            jax.ShapeDtypeStruct((1, K), F32),
        ],
    )(lab3, features)

    p, xxc, xxr = pl.pallas_call(
        _norm_body,
        out_shape=[
            jax.ShapeDtypeStruct((K, D), F32),
            jax.ShapeDtypeStruct((K, 1), F32),
            jax.ShapeDtypeStruct((1, K), F32),
        ],
    )(center_sum, cntc)

    acc = pl.pallas_call(
        _pair_body,
        grid=(K // PB, K // PB),
        in_specs=[
            pl.BlockSpec((PB, D), lambda i, j: (i, 0)),
            pl.BlockSpec((PB, D), lambda i, j: (j, 0)),
            pl.BlockSpec((PB, 1), lambda i, j: (i, 0)),
            pl.BlockSpec((1, PB), lambda i, j: (0, j)),
            pl.BlockSpec((PB, 1), lambda i, j: (i, 0)),
            pl.BlockSpec((1, PB), lambda i, j: (0, j)),
            pl.BlockSpec((1, K), lambda i, j: (0, 0)),
        ],
        out_specs=pl.BlockSpec((1, 128), lambda i, j: (0, 0)),
        out_shape=jax.ShapeDtypeStruct((1, 128), F32),
    )(p, p, xxc, xxr, cntc, cntr, cntr)

    acc2 = pl.pallas_call(
        _tight_body,
        grid=(NBLK,),
        in_specs=[
            pl.BlockSpec((None, 1, RB), lambda i: (i, 0, 0)),
            pl.BlockSpec((None, 1, RB), lambda i: (i, 0, 0)),
            pl.BlockSpec((RB, D), lambda i: (i, 0)),
            pl.BlockSpec((K, D), lambda i: (0, 0)),
        ],
        out_specs=pl.BlockSpec((1, 128), lambda i: (0, 0)),
        out_shape=jax.ShapeDtypeStruct((1, 128), F32),
    )(lab3, pred3, features, p)

    s0 = acc[0, 0]
    s1 = acc[0, 1]
    wmn = acc[0, 2]
    wmx = acc[0, 3]
    npres = acc[0, 4]
    ssum = acc2[0, 0]
    scnt = acc2[0, 1]
    sse = acc2[0, 2]

    n_pairs = npres * (npres - 1.0) * 0.5
    entropy = (s1 - wmn * s0) / wmx / n_pairs
    tight = ssum / jnp.maximum(scnt, 1.0)
    mse = sse / N
    return mse + 0.001 * (tight - entropy)


# speedup vs baseline: 82.9285x; 1.3141x over previous
import jax
import jax.numpy as jnp
from jax import lax
from jax.experimental import pallas as pl
from jax.experimental.pallas import tpu as pltpu

N = 8192
D = 2048
K = 1024
RB = 512
NBLK = N // RB
F32 = jnp.float32
BF16 = jnp.bfloat16


def _fused_body(lab_ref, pred_ref, f_ref, acc_ref, csum_ref, cntc_ref,
                cntr_ref, pb_ref):
    i = pl.program_id(0)
    lane = lax.broadcasted_iota(jnp.int32, (1, 128), 1)

    @pl.when(i == 0)
    def _():
        acc_ref[...] = jnp.where(lane == 2, jnp.inf,
                                 jnp.where(lane == 3, -jnp.inf, 0.0))
        csum_ref[...] = jnp.zeros_like(csum_ref)
        cntc_ref[...] = jnp.zeros_like(cntc_ref)
        cntr_ref[...] = jnp.zeros_like(cntr_ref)

    @pl.when(i < NBLK)
    def _():
        lab_f = lab_ref[...]
        lab = lab_f.astype(jnp.int32)
        ohT = (lax.broadcasted_iota(jnp.int32, (K, RB), 0) == lab)
        ohTb = ohT.astype(BF16)
        csum_ref[...] += lax.dot_general(
            ohTb, f_ref[...].astype(BF16), (((1,), (0,)), ((), ())),
            preferred_element_type=F32)
        ohTf = ohT.astype(F32)
        cntc_ref[...] += jnp.sum(ohTf, axis=1, keepdims=True)
        ones = jnp.ones((1, RB), F32)
        cntr_ref[...] += lax.dot_general(
            ones, ohTf, (((1,), (1,)), ((), ())),
            preferred_element_type=F32)
        e = lab_f - pred_ref[...]
        acc_ref[...] += jnp.where(lane == 7, jnp.sum(e * e), 0.0)

    @pl.when(i == NBLK)
    def _():
        cnt = cntc_ref[...]
        c = csum_ref[...] / jnp.where(cnt > 0, cnt, 1.0)
        nrm = jnp.maximum(jnp.sqrt(jnp.sum(c * c, axis=1, keepdims=True)),
                          1e-12)
        p = c / nrm
        xxc = jnp.sum(p * p, axis=1, keepdims=True)
        psq = p * p
        ones = jnp.ones((1, D), F32)
        xxr = lax.dot_general(ones, psq, (((1,), (1,)), ((), ())),
                              preferred_element_type=F32)
        pb = p.astype(BF16)
        pb_ref[...] = pb
        g = lax.dot_general(pb, pb, (((1,), (1,)), ((), ())),
                            preferred_element_type=F32)
        d = xxc + xxr - 2.0 * g
        dist = jnp.sqrt(jnp.maximum(d, 1e-12))
        gi = lax.broadcasted_iota(jnp.int32, (K, K), 0)
        gj = lax.broadcasted_iota(jnp.int32, (K, K), 1)
        wm = jnp.abs(gi - gj).astype(F32)
        mask = (gj > gi) & (cntc_ref[...] > 0) & (cntr_ref[...] > 0)
        s0 = jnp.sum(jnp.where(mask, dist, 0.0))
        s1 = jnp.sum(jnp.where(mask, dist * wm, 0.0))
        wmn = jnp.min(jnp.where(mask, wm, jnp.inf))
        wmx = jnp.max(jnp.where(mask, wm, -jnp.inf))
        npres = jnp.sum((cntr_ref[...] > 0).astype(F32))
        r = acc_ref[...]
        r = (r + jnp.where(lane == 0, s0, 0.0)
             + jnp.where(lane == 1, s1, 0.0)
             + jnp.where(lane == 4, npres, 0.0))
        r = jnp.where(lane == 2, jnp.minimum(r, wmn), r)
        r = jnp.where(lane == 3, jnp.maximum(r, wmx), r)
        acc_ref[...] = r

    @pl.when(i > NBLK)
    def _():
        lab = lab_ref[...].astype(jnp.int32)
        ohTb = (lax.broadcasted_iota(jnp.int32, (K, RB), 0) == lab).astype(BF16)
        fc = lax.dot_general(ohTb, pb_ref[...],
                             (((0,), (0,)), ((), ())),
                             preferred_element_type=F32)
        dif = f_ref[...] - fc
        t = jnp.sum(dif * dif, axis=1, keepdims=True)
        mask = t > 0
        s = jnp.sqrt(jnp.where(mask, t, 1.0))
        ssum = jnp.sum(jnp.where(mask, s, 0.0))
        scnt = jnp.sum(mask.astype(F32))
        acc_ref[...] += (jnp.where(lane == 5, ssum, 0.0)
                         + jnp.where(lane == 6, scnt, 0.0))


def kernel(features, labels, preds):
    lab3 = labels.reshape(NBLK, 1, RB)
    pred3 = preds.reshape(NBLK, 1, RB)

    def fmap(i):
        return (jnp.where(i < NBLK, i, jnp.maximum(i - (NBLK + 1), 0)), 0)

    def lmap(i):
        return (jnp.where(i < NBLK, i, jnp.maximum(i - (NBLK + 1), 0)), 0, 0)

    acc = pl.pallas_call(
        _fused_body,
        grid=(2 * NBLK + 1,),
        in_specs=[
            pl.BlockSpec((None, 1, RB), lmap),
            pl.BlockSpec((None, 1, RB), lmap),
            pl.BlockSpec((RB, D), fmap),
        ],
        out_specs=pl.BlockSpec((1, 128), lambda i: (0, 0)),
        out_shape=jax.ShapeDtypeStruct((1, 128), F32),
        scratch_shapes=[
            pltpu.VMEM((K, D), F32),
            pltpu.VMEM((K, 1), F32),
            pltpu.VMEM((1, K), F32),
            pltpu.VMEM((K, D), BF16),
        ],
    )(lab3, pred3, features)

    s0 = acc[0, 0]
    s1 = acc[0, 1]
    wmn = acc[0, 2]
    wmx = acc[0, 3]
    npres = acc[0, 4]
    ssum = acc[0, 5]
    scnt = acc[0, 6]
    sse = acc[0, 7]

    n_pairs = npres * (npres - 1.0) * 0.5
    entropy = (s1 - wmn * s0) / wmx / n_pairs
    tight = ssum / jnp.maximum(scnt, 1.0)
    mse = sse / N
    return mse + 0.001 * (tight - entropy)


# speedup vs baseline: 85.4957x; 1.0310x over previous
import jax
import jax.numpy as jnp
from jax import lax
from jax.experimental import pallas as pl
from jax.experimental.pallas import tpu as pltpu

N = 8192
D = 2048
K = 1024
RB = 1024
NBLK = N // RB
F32 = jnp.float32
BF16 = jnp.bfloat16


def _fused_body(lab_ref, pred_ref, f_ref, acc_ref, csum_ref, cntc_ref,
                cntr_ref, pb_ref):
    i = pl.program_id(0)
    lane = lax.broadcasted_iota(jnp.int32, (1, 128), 1)

    @pl.when(i == 0)
    def _():
        acc_ref[...] = jnp.where(lane == 2, jnp.inf,
                                 jnp.where(lane == 3, -jnp.inf, 0.0))
        csum_ref[...] = jnp.zeros_like(csum_ref)
        cntc_ref[...] = jnp.zeros_like(cntc_ref)
        cntr_ref[...] = jnp.zeros_like(cntr_ref)

    @pl.when(i < NBLK)
    def _():
        lab_f = lab_ref[...]
        lab = lab_f.astype(jnp.int32)
        ohT = (lax.broadcasted_iota(jnp.int32, (K, RB), 0) == lab)
        ohTb = ohT.astype(BF16)
        csum_ref[...] += lax.dot_general(
            ohTb, f_ref[...].astype(BF16), (((1,), (0,)), ((), ())),
            preferred_element_type=F32)
        ohTf = ohT.astype(F32)
        cntc_ref[...] += jnp.sum(ohTf, axis=1, keepdims=True)
        ones = jnp.ones((1, RB), F32)
        cntr_ref[...] += lax.dot_general(
            ones, ohTf, (((1,), (1,)), ((), ())),
            preferred_element_type=F32)
        e = lab_f - pred_ref[...]
        acc_ref[...] += jnp.where(lane == 7, jnp.sum(e * e), 0.0)

    @pl.when(i == NBLK)
    def _():
        cnt = cntc_ref[...]
        c = csum_ref[...] / jnp.where(cnt > 0, cnt, 1.0)
        nrm = jnp.maximum(jnp.sqrt(jnp.sum(c * c, axis=1, keepdims=True)),
                          1e-12)
        p = c / nrm
        xxc = jnp.sum(p * p, axis=1, keepdims=True)
        psq = p * p
        ones = jnp.ones((1, D), F32)
        xxr = lax.dot_general(ones, psq, (((1,), (1,)), ((), ())),
                              preferred_element_type=F32)
        pb = p.astype(BF16)
        pb_ref[...] = pb
        g = lax.dot_general(pb, pb, (((1,), (1,)), ((), ())),
                            preferred_element_type=F32)
        d = xxc + xxr - 2.0 * g
        dist = jnp.sqrt(jnp.maximum(d, 1e-12))
        gi = lax.broadcasted_iota(jnp.int32, (K, K), 0)
        gj = lax.broadcasted_iota(jnp.int32, (K, K), 1)
        wm = jnp.abs(gi - gj).astype(F32)
        mask = (gj > gi) & (cntc_ref[...] > 0) & (cntr_ref[...] > 0)
        s0 = jnp.sum(jnp.where(mask, dist, 0.0))
        s1 = jnp.sum(jnp.where(mask, dist * wm, 0.0))
        wmn = jnp.min(jnp.where(mask, wm, jnp.inf))
        wmx = jnp.max(jnp.where(mask, wm, -jnp.inf))
        npres = jnp.sum((cntr_ref[...] > 0).astype(F32))
        r = acc_ref[...]
        r = (r + jnp.where(lane == 0, s0, 0.0)
             + jnp.where(lane == 1, s1, 0.0)
             + jnp.where(lane == 4, npres, 0.0))
        r = jnp.where(lane == 2, jnp.minimum(r, wmn), r)
        r = jnp.where(lane == 3, jnp.maximum(r, wmx), r)
        acc_ref[...] = r

    @pl.when(i > NBLK)
    def _():
        lab = lab_ref[...].astype(jnp.int32)
        ohTb = (lax.broadcasted_iota(jnp.int32, (K, RB), 0) == lab).astype(BF16)
        fc = lax.dot_general(ohTb, pb_ref[...],
                             (((0,), (0,)), ((), ())),
                             preferred_element_type=F32)
        dif = f_ref[...] - fc
        t = jnp.sum(dif * dif, axis=1, keepdims=True)
        mask = t > 0
        s = jnp.sqrt(jnp.where(mask, t, 1.0))
        ssum = jnp.sum(jnp.where(mask, s, 0.0))
        scnt = jnp.sum(mask.astype(F32))
        acc_ref[...] += (jnp.where(lane == 5, ssum, 0.0)
                         + jnp.where(lane == 6, scnt, 0.0))


def kernel(features, labels, preds):
    lab3 = labels.reshape(NBLK, 1, RB)
    pred3 = preds.reshape(NBLK, 1, RB)

    def fmap(i):
        return (jnp.where(i < NBLK, i, jnp.maximum(i - (NBLK + 1), 0)), 0)

    def lmap(i):
        return (jnp.where(i < NBLK, i, jnp.maximum(i - (NBLK + 1), 0)), 0, 0)

    acc = pl.pallas_call(
        _fused_body,
        grid=(2 * NBLK + 1,),
        in_specs=[
            pl.BlockSpec((None, 1, RB), lmap),
            pl.BlockSpec((None, 1, RB), lmap),
            pl.BlockSpec((RB, D), fmap),
        ],
        out_specs=pl.BlockSpec((1, 128), lambda i: (0, 0)),
        out_shape=jax.ShapeDtypeStruct((1, 128), F32),
        scratch_shapes=[
            pltpu.VMEM((K, D), F32),
            pltpu.VMEM((K, 1), F32),
            pltpu.VMEM((1, K), F32),
            pltpu.VMEM((K, D), BF16),
        ],
    )(lab3, pred3, features)

    s0 = acc[0, 0]
    s1 = acc[0, 1]
    wmn = acc[0, 2]
    wmx = acc[0, 3]
    npres = acc[0, 4]
    ssum = acc[0, 5]
    scnt = acc[0, 6]
    sse = acc[0, 7]

    n_pairs = npres * (npres - 1.0) * 0.5
    entropy = (s1 - wmn * s0) / wmx / n_pairs
    tight = ssum / jnp.maximum(scnt, 1.0)
    mse = sse / N
    return mse + 0.001 * (tight - entropy)
